# Initial kernel scaffold; baseline (speedup 1.0000x reference)
import jax
import jax.numpy as jnp
from jax.experimental import pallas as pl


def kernel(mesh_nfeat, edge_index, mesh_efeat, emb_W0, emb_b0, emb_W1, emb_b1, emb_g, emb_bt, We0, be0, We1, be1, eg, ebt, Wn0, bn0, Wn1, bn1, ng, nbt):
    raise NotImplementedError("write your pallas kernel here")


# speedup vs baseline: 2.5104x; 2.5104x over previous
import functools

import jax
import jax.numpy as jnp
from jax import lax
from jax.experimental import pallas as pl
from jax.experimental.pallas import tpu as pltpu
from jax.experimental.pallas import tpu_sc as plsc

N_NODES = 10000
N_EDGES = 160000
D = 128
N_BLOCK = 4

_NC = 2
_NS = 16
_NW = _NC * _NS

E_CHUNK = 128
N_CHUNKS = N_EDGES // E_CHUNK
CPW = -(-N_CHUNKS // _NW)
ZC = 80
NZ = N_NODES // ZC
ZPW = -(-NZ // _NS)

_mesh = plsc.VectorSubcoreMesh(core_axis_name="c", subcore_axis_name="s")


@functools.partial(
    pl.kernel,
    out_type=(jax.ShapeDtypeStruct((N_EDGES, D), jnp.float32),
              jax.ShapeDtypeStruct((N_EDGES, D), jnp.float32)),
    mesh=_mesh,
    scratch_types=[
        pltpu.VMEM((E_CHUNK,), jnp.int32),
        pltpu.VMEM((E_CHUNK,), jnp.int32),
        pltpu.VMEM((E_CHUNK, D), jnp.float32),
        pltpu.VMEM((E_CHUNK, D), jnp.float32),
        pltpu.SemaphoreType.DMA,
        pltpu.SemaphoreType.DMA,
    ],
)
def _sc_gather2(p_hbm, q_hbm, src_hbm, dst_hbm, gs_hbm, gd_hbm,
                idx_s, idx_d, buf_s, buf_d, sem_s, sem_d):
    wid = lax.axis_index("s") * _NC + lax.axis_index("c")

    def body(k, carry):
        ci = k * _NW + wid

        @pl.when(ci < N_CHUNKS)
        def _():
            base = pl.multiple_of(ci * E_CHUNK, E_CHUNK)
            pltpu.sync_copy(src_hbm.at[pl.ds(base, E_CHUNK)], idx_s)
            pltpu.sync_copy(dst_hbm.at[pl.ds(base, E_CHUNK)], idx_d)
            cps = pltpu.async_copy(p_hbm.at[idx_s], buf_s, sem_s)
            cpd = pltpu.async_copy(q_hbm.at[idx_d], buf_d, sem_d)
            cps.wait()
            cpd.wait()
            pltpu.sync_copy(buf_s, gs_hbm.at[pl.ds(base, E_CHUNK)])
            pltpu.sync_copy(buf_d, gd_hbm.at[pl.ds(base, E_CHUNK)])

        return carry

    lax.fori_loop(0, CPW, body, 0)


@functools.partial(
    pl.kernel,
    out_type=jax.ShapeDtypeStruct((_NC, N_NODES, D), jnp.float32),
    mesh=_mesh,
    scratch_types=[
        pltpu.VMEM_SHARED((N_NODES, D), jnp.float32),
        pltpu.VMEM((E_CHUNK, D), jnp.float32),
        pltpu.VMEM((E_CHUNK,), jnp.int32),
    ],
)
def _sc_scatter(e_hbm, dst_hbm, out_hbm, acc_sh, buf_e, idx_d):
    cid = lax.axis_index("c")
    sid = lax.axis_index("s")
    wid = sid * _NC + cid

    def zb(i, carry):
        r = i // (D // 16)
        c2 = (i % (D // 16)) * 16
        buf_e[r, pl.ds(c2, 16)] = jnp.zeros((16,), jnp.float32)
        return carry

    lax.fori_loop(0, E_CHUNK * (D // 16), zb, 0)

    def zc(z, carry):
        ci = z * _NS + sid

        @pl.when(ci < NZ)
        def _():
            base = pl.multiple_of(ci * ZC, 8)
            pltpu.sync_copy(buf_e.at[pl.ds(0, ZC)], acc_sh.at[pl.ds(base, ZC)])

        return carry

    lax.fori_loop(0, ZPW, zc, 0)
    plsc.subcore_barrier()

    def body(k, carry):
        ci = k * _NW + wid

        @pl.when(ci < N_CHUNKS)
        def _():
            base = pl.multiple_of(ci * E_CHUNK, E_CHUNK)
            pltpu.sync_copy(e_hbm.at[pl.ds(base, E_CHUNK)], buf_e)
            pltpu.sync_copy(dst_hbm.at[pl.ds(base, E_CHUNK)], idx_d)
            pltpu.sync_copy(buf_e, acc_sh.at[idx_d], add=True)

        return carry

    lax.fori_loop(0, CPW, body, 0)
    plsc.subcore_barrier()

    def oc(z, carry):
        ci = z * _NS + sid

        @pl.when(ci < NZ)
        def _():
            base = pl.multiple_of(ci * ZC, 8)
            pltpu.sync_copy(acc_sh.at[pl.ds(base, ZC)],
                            out_hbm.at[cid, pl.ds(base, ZC)])

        return carry

    lax.fori_loop(0, ZPW, oc, 0)


def _ln(y, g, bt):
    mu = jnp.mean(y, axis=-1, keepdims=True)
    var = jnp.mean((y - mu) * (y - mu), axis=-1, keepdims=True)
    return (y - mu) * lax.rsqrt(var + 1e-5) * g + bt


def _embed_body(x_ref, w0, b0, w1, b1, g, bt, o_ref):
    h = jax.nn.silu(jnp.dot(x_ref[...], w0[...],
                            preferred_element_type=jnp.float32) + b0[...])
    y = jnp.dot(h, w1[...], preferred_element_type=jnp.float32) + b1[...]
    o_ref[...] = _ln(y, g[...], bt[...])


def _pq_body(n_ref, ws, wd, p_ref, q_ref):
    x = n_ref[...]
    p_ref[...] = jnp.dot(x, ws[...], preferred_element_type=jnp.float32)
    q_ref[...] = jnp.dot(x, wd[...], preferred_element_type=jnp.float32)


def _edge_body(gs_ref, gd_ref, e_ref, we, b0, w1, b1, g, bt, o_ref):
    x = e_ref[...]
    pre = gs_ref[...] + gd_ref[...] + jnp.dot(
        x, we[...], preferred_element_type=jnp.float32) + b0[...]
    h = jax.nn.silu(pre)
    y = jnp.dot(h, w1[...], preferred_element_type=jnp.float32) + b1[...]
    o_ref[...] = x + _ln(y, g[...], bt[...])


def _node_body(a_ref, n_ref, wa, wn, b0, w1, b1, g, bt, o_ref):
    agg = a_ref[0] + a_ref[1]
    x = n_ref[...]
    pre = (jnp.dot(agg, wa[...], preferred_element_type=jnp.float32)
           + jnp.dot(x, wn[...], preferred_element_type=jnp.float32)
           + b0[...])
    h = jax.nn.silu(pre)
    y = jnp.dot(h, w1[...], preferred_element_type=jnp.float32) + b1[...]
    o_ref[...] = x + _ln(y, g[...], bt[...])


_E_TILE = 1000
_N_TILE = 1000


def _full(shape):
    return pl.BlockSpec(shape, lambda i: (0,) * len(shape))


def _tc_embed(x, w0, b0, w1, b1, g, bt):
    grid = (N_EDGES // _E_TILE,)
    return pl.pallas_call(
        _embed_body,
        grid=grid,
        in_specs=[
            pl.BlockSpec((_E_TILE, 4), lambda i: (i, 0)),
            _full((4, D)), _full((1, D)), _full((D, D)), _full((1, D)),
            _full((1, D)), _full((1, D)),
        ],
        out_specs=pl.BlockSpec((_E_TILE, D), lambda i: (i, 0)),
        out_shape=jax.ShapeDtypeStruct((N_EDGES, D), jnp.float32),
    )(x, w0, b0, w1, b1, g, bt)


def _tc_pq(nfeat, ws, wd):
    grid = (N_NODES // _N_TILE,)
    return pl.pallas_call(
        _pq_body,
        grid=grid,
        in_specs=[
            pl.BlockSpec((_N_TILE, D), lambda i: (i, 0)),
            _full((D, D)), _full((D, D)),
        ],
        out_specs=[pl.BlockSpec((_N_TILE, D), lambda i: (i, 0)),
                   pl.BlockSpec((_N_TILE, D), lambda i: (i, 0))],
        out_shape=[jax.ShapeDtypeStruct((N_NODES, D), jnp.float32),
                   jax.ShapeDtypeStruct((N_NODES, D), jnp.float32)],
    )(nfeat, ws, wd)


def _tc_edge(gs, gd, efeat, we, b0, w1, b1, g, bt):
    grid = (N_EDGES // _E_TILE,)
    return pl.pallas_call(
        _edge_body,
        grid=grid,
        in_specs=[
            pl.BlockSpec((_E_TILE, D), lambda i: (i, 0)),
            pl.BlockSpec((_E_TILE, D), lambda i: (i, 0)),
            pl.BlockSpec((_E_TILE, D), lambda i: (i, 0)),
            _full((D, D)), _full((1, D)), _full((D, D)), _full((1, D)),
            _full((1, D)), _full((1, D)),
        ],
        out_specs=pl.BlockSpec((_E_TILE, D), lambda i: (i, 0)),
        out_shape=jax.ShapeDtypeStruct((N_EDGES, D), jnp.float32),
    )(gs, gd, efeat, we, b0, w1, b1, g, bt)


def _tc_node(a2, nfeat, wa, wn, b0, w1, b1, g, bt):
    grid = (N_NODES // _N_TILE,)
    return pl.pallas_call(
        _node_body,
        grid=grid,
        in_specs=[
            pl.BlockSpec((_NC, _N_TILE, D), lambda i: (0, i, 0)),
            pl.BlockSpec((_N_TILE, D), lambda i: (i, 0)),
            _full((D, D)), _full((D, D)), _full((1, D)), _full((D, D)),
            _full((1, D)), _full((1, D)), _full((1, D)),
        ],
        out_specs=pl.BlockSpec((_N_TILE, D), lambda i: (i, 0)),
        out_shape=jax.ShapeDtypeStruct((N_NODES, D), jnp.float32),
    )(a2, nfeat, wa, wn, b0, w1, b1, g, bt)


def kernel(mesh_nfeat, edge_index, mesh_efeat,
           emb_W0, emb_b0, emb_W1, emb_b1, emb_g, emb_bt,
           We0, be0, We1, be1, eg, ebt,
           Wn0, bn0, Wn1, bn1, ng, nbt):
    r = lambda v: v.reshape(1, D)
    src = edge_index[0].astype(jnp.int32)
    dst = edge_index[1].astype(jnp.int32)

    efeat = _tc_embed(mesh_efeat, emb_W0, r(emb_b0), emb_W1, r(emb_b1),
                      r(emb_g), r(emb_bt))
    nfeat = mesh_nfeat
    for i in range(N_BLOCK):
        p, q = _tc_pq(nfeat, We0[i, :D], We0[i, D:2 * D])
        gs, gd = _sc_gather2(p, q, src, dst)
        efeat = _tc_edge(gs, gd, efeat, We0[i, 2 * D:], r(be0[i]),
                         We1[i], r(be1[i]), r(eg[i]), r(ebt[i]))
        a2 = _sc_scatter(efeat, dst)
        nfeat = _tc_node(a2, nfeat, Wn0[i, :D], Wn0[i, D:], r(bn0[i]),
                         Wn1[i], r(bn1[i]), r(ng[i]), r(nbt[i]))
    return (nfeat, efeat)


# speedup vs baseline: 2.6180x; 1.0429x over previous
import functools

import jax
import jax.numpy as jnp
from jax import lax
from jax.experimental import pallas as pl
from jax.experimental.pallas import tpu as pltpu
from jax.experimental.pallas import tpu_sc as plsc

N_NODES = 10000
N_EDGES = 160000
D = 128
N_BLOCK = 4

_NC = 2
_NS = 16
_NW = _NC * _NS

E_CHUNK = 128
N_CHUNKS = N_EDGES // E_CHUNK
CPW = -(-N_CHUNKS // _NW)
ZC = 80
NZ = N_NODES // ZC
ZPW = -(-NZ // _NS)

_mesh = plsc.VectorSubcoreMesh(core_axis_name="c", subcore_axis_name="s")


CPS = -(-N_CHUNKS // _NS)


@functools.partial(
    pl.kernel,
    out_type=(jax.ShapeDtypeStruct((N_EDGES, D), jnp.float32),
              jax.ShapeDtypeStruct((N_EDGES, D), jnp.float32)),
    mesh=_mesh,
    scratch_types=[
        pltpu.VMEM_SHARED((N_NODES, D), jnp.float32),
        pltpu.VMEM((E_CHUNK,), jnp.int32),
        pltpu.VMEM((E_CHUNK, D), jnp.float32),
        pltpu.SemaphoreType.DMA,
    ],
)
def _sc_gather2(p_hbm, q_hbm, src_hbm, dst_hbm, gs_hbm, gd_hbm,
                tbl_sh, idx_b, buf, sem):
    cid = lax.axis_index("c")
    sid = lax.axis_index("s")

    def stage(tab_hbm):
        def st(z, carry):
            ci = z * _NS + sid

            @pl.when(ci < NZ)
            def _():
                base = pl.multiple_of(ci * ZC, 8)
                pltpu.sync_copy(tab_hbm.at[pl.ds(base, ZC)],
                                tbl_sh.at[pl.ds(base, ZC)])

            return carry

        lax.fori_loop(0, ZPW, st, 0)

    @pl.when(cid == 0)
    def _():
        stage(p_hbm)

    @pl.when(cid == 1)
    def _():
        stage(q_hbm)

    plsc.subcore_barrier()

    def run(ind_hbm, out_hbm):
        def body(k, carry):
            ci = k * _NS + sid

            @pl.when(ci < N_CHUNKS)
            def _():
                base = pl.multiple_of(ci * E_CHUNK, E_CHUNK)
                pltpu.sync_copy(ind_hbm.at[pl.ds(base, E_CHUNK)], idx_b)
                pltpu.async_copy(tbl_sh.at[idx_b], buf, sem).wait()
                pltpu.sync_copy(buf, out_hbm.at[pl.ds(base, E_CHUNK)])

            return carry

        lax.fori_loop(0, CPS, body, 0)

    @pl.when(cid == 0)
    def _():
        run(src_hbm, gs_hbm)

    @pl.when(cid == 1)
    def _():
        run(dst_hbm, gd_hbm)


@functools.partial(
    pl.kernel,
    out_type=jax.ShapeDtypeStruct((_NC, N_NODES, D), jnp.float32),
    mesh=_mesh,
    scratch_types=[
        pltpu.VMEM_SHARED((N_NODES, D), jnp.float32),
        pltpu.VMEM((E_CHUNK, D), jnp.float32),
        pltpu.VMEM((E_CHUNK,), jnp.int32),
    ],
)
def _sc_scatter(e_hbm, dst_hbm, out_hbm, acc_sh, buf_e, idx_d):
    cid = lax.axis_index("c")
    sid = lax.axis_index("s")
    wid = sid * _NC + cid

    def zb(i, carry):
        r = i // (D // 16)
        c2 = (i % (D // 16)) * 16
        buf_e[r, pl.ds(c2, 16)] = jnp.zeros((16,), jnp.float32)
        return carry

    lax.fori_loop(0, E_CHUNK * (D // 16), zb, 0)

    def zc(z, carry):
        ci = z * _NS + sid

        @pl.when(ci < NZ)
        def _():
            base = pl.multiple_of(ci * ZC, 8)
            pltpu.sync_copy(buf_e.at[pl.ds(0, ZC)], acc_sh.at[pl.ds(base, ZC)])

        return carry

    lax.fori_loop(0, ZPW, zc, 0)
    plsc.subcore_barrier()

    def body(k, carry):
        ci = k * _NW + wid

        @pl.when(ci < N_CHUNKS)
        def _():
            base = pl.multiple_of(ci * E_CHUNK, E_CHUNK)
            pltpu.sync_copy(e_hbm.at[pl.ds(base, E_CHUNK)], buf_e)
            pltpu.sync_copy(dst_hbm.at[pl.ds(base, E_CHUNK)], idx_d)
            pltpu.sync_copy(buf_e, acc_sh.at[idx_d], add=True)

        return carry

    lax.fori_loop(0, CPW, body, 0)
    plsc.subcore_barrier()

    def oc(z, carry):
        ci = z * _NS + sid

        @pl.when(ci < NZ)
        def _():
            base = pl.multiple_of(ci * ZC, 8)
            pltpu.sync_copy(acc_sh.at[pl.ds(base, ZC)],
                            out_hbm.at[cid, pl.ds(base, ZC)])

        return carry

    lax.fori_loop(0, ZPW, oc, 0)


def _ln(y, g, bt):
    mu = jnp.mean(y, axis=-1, keepdims=True)
    var = jnp.mean((y - mu) * (y - mu), axis=-1, keepdims=True)
    return (y - mu) * lax.rsqrt(var + 1e-5) * g + bt


def _embed_body(x_ref, w0, b0, w1, b1, g, bt, o_ref):
    h = jax.nn.silu(jnp.dot(x_ref[...], w0[...],
                            preferred_element_type=jnp.float32) + b0[...])
    y = jnp.dot(h, w1[...], preferred_element_type=jnp.float32) + b1[...]
    o_ref[...] = _ln(y, g[...], bt[...])


def _pq_body(n_ref, ws, wd, p_ref, q_ref):
    x = n_ref[...]
    p_ref[...] = jnp.dot(x, ws[...], preferred_element_type=jnp.float32)
    q_ref[...] = jnp.dot(x, wd[...], preferred_element_type=jnp.float32)


def _edge_body(gs_ref, gd_ref, e_ref, we, b0, w1, b1, g, bt, o_ref):
    x = e_ref[...]
    pre = gs_ref[...] + gd_ref[...] + jnp.dot(
        x, we[...], preferred_element_type=jnp.float32) + b0[...]
    h = jax.nn.silu(pre)
    y = jnp.dot(h, w1[...], preferred_element_type=jnp.float32) + b1[...]
    o_ref[...] = x + _ln(y, g[...], bt[...])


def _node_body(a_ref, n_ref, wa, wn, b0, w1, b1, g, bt, o_ref):
    agg = a_ref[0] + a_ref[1]
    x = n_ref[...]
    pre = (jnp.dot(agg, wa[...], preferred_element_type=jnp.float32)
           + jnp.dot(x, wn[...], preferred_element_type=jnp.float32)
           + b0[...])
    h = jax.nn.silu(pre)
    y = jnp.dot(h, w1[...], preferred_element_type=jnp.float32) + b1[...]
    o_ref[...] = x + _ln(y, g[...], bt[...])


_E_TILE = 1000
_N_TILE = 1000


def _full(shape):
    return pl.BlockSpec(shape, lambda i: (0,) * len(shape))


def _tc_embed(x, w0, b0, w1, b1, g, bt):
    grid = (N_EDGES // _E_TILE,)
    return pl.pallas_call(
        _embed_body,
        grid=grid,
        in_specs=[
            pl.BlockSpec((_E_TILE, 4), lambda i: (i, 0)),
            _full((4, D)), _full((1, D)), _full((D, D)), _full((1, D)),
            _full((1, D)), _full((1, D)),
        ],
        out_specs=pl.BlockSpec((_E_TILE, D), lambda i: (i, 0)),
        out_shape=jax.ShapeDtypeStruct((N_EDGES, D), jnp.float32),
    )(x, w0, b0, w1, b1, g, bt)


def _tc_pq(nfeat, ws, wd):
    grid = (N_NODES // _N_TILE,)
    return pl.pallas_call(
        _pq_body,
        grid=grid,
        in_specs=[
            pl.BlockSpec((_N_TILE, D), lambda i: (i, 0)),
            _full((D, D)), _full((D, D)),
        ],
        out_specs=[pl.BlockSpec((_N_TILE, D), lambda i: (i, 0)),
                   pl.BlockSpec((_N_TILE, D), lambda i: (i, 0))],
        out_shape=[jax.ShapeDtypeStruct((N_NODES, D), jnp.float32),
                   jax.ShapeDtypeStruct((N_NODES, D), jnp.float32)],
    )(nfeat, ws, wd)


def _tc_edge(gs, gd, efeat, we, b0, w1, b1, g, bt):
    grid = (N_EDGES // _E_TILE,)
    return pl.pallas_call(
        _edge_body,
        grid=grid,
        in_specs=[
            pl.BlockSpec((_E_TILE, D), lambda i: (i, 0)),
            pl.BlockSpec((_E_TILE, D), lambda i: (i, 0)),
            pl.BlockSpec((_E_TILE, D), lambda i: (i, 0)),
            _full((D, D)), _full((1, D)), _full((D, D)), _full((1, D)),
            _full((1, D)), _full((1, D)),
        ],
        out_specs=pl.BlockSpec((_E_TILE, D), lambda i: (i, 0)),
        out_shape=jax.ShapeDtypeStruct((N_EDGES, D), jnp.float32),
    )(gs, gd, efeat, we, b0, w1, b1, g, bt)


def _tc_node(a2, nfeat, wa, wn, b0, w1, b1, g, bt):
    grid = (N_NODES // _N_TILE,)
    return pl.pallas_call(
        _node_body,
        grid=grid,
        in_specs=[
            pl.BlockSpec((_NC, _N_TILE, D), lambda i: (0, i, 0)),
            pl.BlockSpec((_N_TILE, D), lambda i: (i, 0)),
            _full((D, D)), _full((D, D)), _full((1, D)), _full((D, D)),
            _full((1, D)), _full((1, D)), _full((1, D)),
        ],
        out_specs=pl.BlockSpec((_N_TILE, D), lambda i: (i, 0)),
        out_shape=jax.ShapeDtypeStruct((N_NODES, D), jnp.float32),
    )(a2, nfeat, wa, wn, b0, w1, b1, g, bt)


def kernel(mesh_nfeat, edge_index, mesh_efeat,
           emb_W0, emb_b0, emb_W1, emb_b1, emb_g, emb_bt,
           We0, be0, We1, be1, eg, ebt,
           Wn0, bn0, Wn1, bn1, ng, nbt):
    r = lambda v: v.reshape(1, D)
    src = edge_index[0].astype(jnp.int32)
    dst = edge_index[1].astype(jnp.int32)

    efeat = _tc_embed(mesh_efeat, emb_W0, r(emb_b0), emb_W1, r(emb_b1),
                      r(emb_g), r(emb_bt))
    nfeat = mesh_nfeat
    for i in range(N_BLOCK):
        p, q = _tc_pq(nfeat, We0[i, :D], We0[i, D:2 * D])
        gs, gd = _sc_gather2(p, q, src, dst)
        efeat = _tc_edge(gs, gd, efeat, We0[i, 2 * D:], r(be0[i]),
                         We1[i], r(be1[i]), r(eg[i]), r(ebt[i]))
        a2 = _sc_scatter(efeat, dst)
        nfeat = _tc_node(a2, nfeat, Wn0[i, :D], Wn0[i, D:], r(bn0[i]),
                         Wn1[i], r(bn1[i]), r(ng[i]), r(nbt[i]))
    return (nfeat, efeat)


# speedup vs baseline: 3.1493x; 1.2029x over previous
import functools

import jax
import jax.numpy as jnp
from jax import lax
from jax.experimental import pallas as pl
from jax.experimental.pallas import tpu as pltpu
from jax.experimental.pallas import tpu_sc as plsc

N_NODES = 10000
N_EDGES = 160000
D = 128
N_BLOCK = 4

_NC = 2
_NS = 16
_NW = _NC * _NS

E_CHUNK = 128
N_CHUNKS = N_EDGES // E_CHUNK
CPW = -(-N_CHUNKS // _NW)
ZC = 80
NZ = N_NODES // ZC
ZPW = -(-NZ // _NS)

_mesh = plsc.VectorSubcoreMesh(core_axis_name="c", subcore_axis_name="s")


GE = 80
EPT = N_EDGES // _NS
GN = EPT // GE


@functools.partial(
    pl.kernel,
    out_type=(jax.ShapeDtypeStruct((N_EDGES, D), jnp.float32),
              jax.ShapeDtypeStruct((N_EDGES, D), jnp.float32)),
    mesh=_mesh,
    scratch_types=[
        pltpu.VMEM_SHARED((N_NODES, D), jnp.float32),
        pltpu.VMEM((GE,), jnp.int32),
        pltpu.VMEM((GE,), jnp.int32),
        pltpu.VMEM((GE, D), jnp.float32),
        pltpu.VMEM((GE, D), jnp.float32),
        pltpu.SemaphoreType.DMA,
        pltpu.SemaphoreType.DMA,
        pltpu.SemaphoreType.DMA,
        pltpu.SemaphoreType.DMA,
        pltpu.SemaphoreType.DMA,
        pltpu.SemaphoreType.DMA,
    ],
)
def _sc_gather2(p_hbm, q_hbm, src_hbm, dst_hbm, gs_hbm, gd_hbm,
                tbl_sh, idx0, idx1, buf0, buf1,
                isem0, isem1, gsem0, gsem1, wsem0, wsem1):
    cid = lax.axis_index("c")
    sid = lax.axis_index("s")
    idxs = (idx0, idx1)
    bufs = (buf0, buf1)
    isems = (isem0, isem1)
    gsems = (gsem0, gsem1)
    wsems = (wsem0, wsem1)

    def stage(tab_hbm):
        def st(z, carry):
            ci = z * _NS + sid

            @pl.when(ci < NZ)
            def _():
                base = pl.multiple_of(ci * ZC, 8)
                pltpu.sync_copy(tab_hbm.at[pl.ds(base, ZC)],
                                tbl_sh.at[pl.ds(base, ZC)])

            return carry

        lax.fori_loop(0, ZPW, st, 0)

    @pl.when(cid == 0)
    def _():
        stage(p_hbm)

    @pl.when(cid == 1)
    def _():
        stage(q_hbm)

    plsc.subcore_barrier()

    def run(ind_hbm, out_hbm):
        tb = sid * EPT

        def idxload(k, b):
            pltpu.async_copy(ind_hbm.at[pl.ds(tb + k * GE, GE)],
                             idxs[b], isems[b])

        idxload(0, 0)
        idxload(1, 1)
        pltpu.make_async_copy(ind_hbm.at[pl.ds(0, GE)], idx0, isem0).wait()
        pltpu.async_copy(tbl_sh.at[idx0], buf0, gsem0)

        def step(k, b):
            o = 1 - b
            pltpu.make_async_copy(tbl_sh.at[idxs[b]], bufs[b],
                                  gsems[b]).wait()

            @pl.when(k + 2 < GN)
            def _():
                idxload(k + 2, b)

            pltpu.async_copy(bufs[b], out_hbm.at[pl.ds(tb + k * GE, GE)],
                             wsems[b])
            pltpu.make_async_copy(ind_hbm.at[pl.ds(0, GE)], idxs[o],
                                  isems[o]).wait()

            @pl.when(k >= 1)
            def _():
                pltpu.make_async_copy(bufs[o], out_hbm.at[pl.ds(0, GE)],
                                      wsems[o]).wait()

            pltpu.async_copy(tbl_sh.at[idxs[o]], bufs[o], gsems[o])

        def body(k2, carry):
            step(k2 * 2, 0)
            step(k2 * 2 + 1, 1)
            return carry

        lax.fori_loop(0, (GN - 1) // 2, body, 0)
        pltpu.make_async_copy(tbl_sh.at[idx0], buf0, gsem0).wait()
        pltpu.async_copy(buf0, out_hbm.at[pl.ds(tb + (GN - 1) * GE, GE)],
                         wsem0)
        pltpu.make_async_copy(buf1, out_hbm.at[pl.ds(0, GE)], wsem1).wait()
        pltpu.make_async_copy(buf0, out_hbm.at[pl.ds(0, GE)], wsem0).wait()

    @pl.when(cid == 0)
    def _():
        run(src_hbm, gs_hbm)

    @pl.when(cid == 1)
    def _():
        run(dst_hbm, gd_hbm)


SE = 80
EPW = N_EDGES // _NW
SN = EPW // SE
ST = EPW - SN * SE


@functools.partial(
    pl.kernel,
    out_type=jax.ShapeDtypeStruct((_NC, N_NODES, D), jnp.float32),
    mesh=_mesh,
    scratch_types=[
        pltpu.VMEM_SHARED((N_NODES, D), jnp.float32),
        pltpu.VMEM((SE, D), jnp.float32),
        pltpu.VMEM((SE, D), jnp.float32),
        pltpu.VMEM((SE,), jnp.int32),
        pltpu.VMEM((SE,), jnp.int32),
        pltpu.VMEM((ST,), jnp.int32),
        pltpu.SemaphoreType.DMA,
        pltpu.SemaphoreType.DMA,
        pltpu.SemaphoreType.DMA,
        pltpu.SemaphoreType.DMA,
        pltpu.SemaphoreType.DMA,
        pltpu.SemaphoreType.DMA,
    ],
)
def _sc_scatter(e_hbm, dst_hbm, out_hbm, acc_sh, ebuf0, ebuf1,
                idx0, idx1, idx_t,
                esem0, esem1, isem0, isem1, ssem0, ssem1):
    cid = lax.axis_index("c")
    sid = lax.axis_index("s")
    wid = sid * _NC + cid
    wb = wid * EPW
    ebufs = (ebuf0, ebuf1)
    idxs = (idx0, idx1)
    esems = (esem0, esem1)
    isems = (isem0, isem1)
    ssems = (ssem0, ssem1)

    def zb(i, carry):
        r = i // (D // 16)
        c2 = (i % (D // 16)) * 16
        ebuf0[r, pl.ds(c2, 16)] = jnp.zeros((16,), jnp.float32)
        return carry

    lax.fori_loop(0, SE * (D // 16), zb, 0)

    def zc(z, carry):
        ci = z * _NS + sid

        @pl.when(ci < NZ)
        def _():
            base = pl.multiple_of(ci * ZC, 8)
            pltpu.sync_copy(ebuf0, acc_sh.at[pl.ds(base, ZC)])

        return carry

    lax.fori_loop(0, ZPW, zc, 0)
    plsc.subcore_barrier()

    def load(k, b):
        pltpu.async_copy(e_hbm.at[pl.ds(wb + k * SE, SE)], ebufs[b],
                         esems[b])
        pltpu.async_copy(dst_hbm.at[pl.ds(wb + k * SE, SE)], idxs[b],
                         isems[b])

    load(0, 0)

    def step(k, b):
        o = 1 - b
        pltpu.make_async_copy(e_hbm.at[pl.ds(0, SE)], ebufs[b],
                              esems[b]).wait()
        pltpu.make_async_copy(dst_hbm.at[pl.ds(0, SE)], idxs[b],
                              isems[b]).wait()
        pltpu.async_copy(ebufs[b], acc_sh.at[idxs[b]], ssems[b], add=True)

        @pl.when(k >= 1)
        def _():
            pltpu.make_async_copy(ebufs[o], acc_sh.at[idxs[o]],
                                  ssems[o]).wait()

        @pl.when(k + 1 < SN)
        def _():
            load(k + 1, o)

    def body(k2, carry):
        step(k2 * 2, 0)
        step(k2 * 2 + 1, 1)
        return carry

    lax.fori_loop(0, SN // 2, body, 0)
    pltpu.sync_copy(e_hbm.at[pl.ds(wb + SN * SE, ST)],
                    ebuf0.at[pl.ds(0, ST)])
    pltpu.sync_copy(dst_hbm.at[pl.ds(wb + SN * SE, ST)], idx_t)
    pltpu.sync_copy(ebuf0.at[pl.ds(0, ST)], acc_sh.at[idx_t], add=True)
    pltpu.make_async_copy(ebuf1, acc_sh.at[idx1], ssem1).wait()
    plsc.subcore_barrier()

    def oc(z, carry):
        ci = z * _NS + sid

        @pl.when(ci < NZ)
        def _():
            base = pl.multiple_of(ci * ZC, 8)
            pltpu.sync_copy(acc_sh.at[pl.ds(base, ZC)],
                            out_hbm.at[cid, pl.ds(base, ZC)])

        return carry

    lax.fori_loop(0, ZPW, oc, 0)


def _ln(y, g, bt):
    mu = jnp.mean(y, axis=-1, keepdims=True)
    var = jnp.mean((y - mu) * (y - mu), axis=-1, keepdims=True)
    return (y - mu) * lax.rsqrt(var + 1e-5) * g + bt


def _embed_body(x_ref, w0, b0, w1, b1, g, bt, o_ref):
    h = jax.nn.silu(jnp.dot(x_ref[...], w0[...],
                            preferred_element_type=jnp.float32) + b0[...])
    y = jnp.dot(h, w1[...], preferred_element_type=jnp.float32) + b1[...]
    o_ref[...] = _ln(y, g[...], bt[...])


def _pq_body(n_ref, ws, wd, p_ref, q_ref):
    x = n_ref[...]
    p_ref[...] = jnp.dot(x, ws[...], preferred_element_type=jnp.float32)
    q_ref[...] = jnp.dot(x, wd[...], preferred_element_type=jnp.float32)


def _edge_body(gs_ref, gd_ref, e_ref, we, b0, w1, b1, g, bt, o_ref):
    x = e_ref[...]
    pre = gs_ref[...] + gd_ref[...] + jnp.dot(
        x, we[...], preferred_element_type=jnp.float32) + b0[...]
    h = jax.nn.silu(pre)
    y = jnp.dot(h, w1[...], preferred_element_type=jnp.float32) + b1[...]
    o_ref[...] = x + _ln(y, g[...], bt[...])


def _node_body(a_ref, n_ref, wa, wn, b0, w1, b1, g, bt, o_ref):
    agg = a_ref[0] + a_ref[1]
    x = n_ref[...]
    pre = (jnp.dot(agg, wa[...], preferred_element_type=jnp.float32)
           + jnp.dot(x, wn[...], preferred_element_type=jnp.float32)
           + b0[...])
    h = jax.nn.silu(pre)
    y = jnp.dot(h, w1[...], preferred_element_type=jnp.float32) + b1[...]
    o_ref[...] = x + _ln(y, g[...], bt[...])


_E_TILE = 1000
_N_TILE = 1000


def _full(shape):
    return pl.BlockSpec(shape, lambda i: (0,) * len(shape))


def _tc_embed(x, w0, b0, w1, b1, g, bt):
    grid = (N_EDGES // _E_TILE,)
    return pl.pallas_call(
        _embed_body,
        grid=grid,
        in_specs=[
            pl.BlockSpec((_E_TILE, 4), lambda i: (i, 0)),
            _full((4, D)), _full((1, D)), _full((D, D)), _full((1, D)),
            _full((1, D)), _full((1, D)),
        ],
        out_specs=pl.BlockSpec((_E_TILE, D), lambda i: (i, 0)),
        out_shape=jax.ShapeDtypeStruct((N_EDGES, D), jnp.float32),
    )(x, w0, b0, w1, b1, g, bt)


def _tc_pq(nfeat, ws, wd):
    grid = (N_NODES // _N_TILE,)
    return pl.pallas_call(
        _pq_body,
        grid=grid,
        in_specs=[
            pl.BlockSpec((_N_TILE, D), lambda i: (i, 0)),
            _full((D, D)), _full((D, D)),
        ],
        out_specs=[pl.BlockSpec((_N_TILE, D), lambda i: (i, 0)),
                   pl.BlockSpec((_N_TILE, D), lambda i: (i, 0))],
        out_shape=[jax.ShapeDtypeStruct((N_NODES, D), jnp.float32),
                   jax.ShapeDtypeStruct((N_NODES, D), jnp.float32)],
    )(nfeat, ws, wd)


def _tc_edge(gs, gd, efeat, we, b0, w1, b1, g, bt):
    grid = (N_EDGES // _E_TILE,)
    return pl.pallas_call(
        _edge_body,
        grid=grid,
        in_specs=[
            pl.BlockSpec((_E_TILE, D), lambda i: (i, 0)),
            pl.BlockSpec((_E_TILE, D), lambda i: (i, 0)),
            pl.BlockSpec((_E_TILE, D), lambda i: (i, 0)),
            _full((D, D)), _full((1, D)), _full((D, D)), _full((1, D)),
            _full((1, D)), _full((1, D)),
        ],
        out_specs=pl.BlockSpec((_E_TILE, D), lambda i: (i, 0)),
        out_shape=jax.ShapeDtypeStruct((N_EDGES, D), jnp.float32),
    )(gs, gd, efeat, we, b0, w1, b1, g, bt)


def _tc_node(a2, nfeat, wa, wn, b0, w1, b1, g, bt):
    grid = (N_NODES // _N_TILE,)
    return pl.pallas_call(
        _node_body,
        grid=grid,
        in_specs=[
            pl.BlockSpec((_NC, _N_TILE, D), lambda i: (0, i, 0)),
            pl.BlockSpec((_N_TILE, D), lambda i: (i, 0)),
            _full((D, D)), _full((D, D)), _full((1, D)), _full((D, D)),
            _full((1, D)), _full((1, D)), _full((1, D)),
        ],
        out_specs=pl.BlockSpec((_N_TILE, D), lambda i: (i, 0)),
        out_shape=jax.ShapeDtypeStruct((N_NODES, D), jnp.float32),
    )(a2, nfeat, wa, wn, b0, w1, b1, g, bt)


def kernel(mesh_nfeat, edge_index, mesh_efeat,
           emb_W0, emb_b0, emb_W1, emb_b1, emb_g, emb_bt,
           We0, be0, We1, be1, eg, ebt,
           Wn0, bn0, Wn1, bn1, ng, nbt):
    r = lambda v: v.reshape(1, D)
    src = edge_index[0].astype(jnp.int32)
    dst = edge_index[1].astype(jnp.int32)

    efeat = _tc_embed(mesh_efeat, emb_W0, r(emb_b0), emb_W1, r(emb_b1),
                      r(emb_g), r(emb_bt))
    nfeat = mesh_nfeat
    for i in range(N_BLOCK):
        p, q = _tc_pq(nfeat, We0[i, :D], We0[i, D:2 * D])
        gs, gd = _sc_gather2(p, q, src, dst)
        efeat = _tc_edge(gs, gd, efeat, We0[i, 2 * D:], r(be0[i]),
                         We1[i], r(be1[i]), r(eg[i]), r(ebt[i]))
        a2 = _sc_scatter(efeat, dst)
        nfeat = _tc_node(a2, nfeat, Wn0[i, :D], Wn0[i, D:], r(bn0[i]),
                         Wn1[i], r(bn1[i]), r(ng[i]), r(nbt[i]))
    return (nfeat, efeat)
